# baseline (speedup 1.0000x reference)
import jax
import jax.numpy as jnp
from jax import lax
from jax.experimental import pallas as pl
from jax.experimental.pallas import tpu as pltpu
from jax.experimental.pallas import tpu_sc as plsc

N = 50000
R = 8
RT = 2 * R + 1
EMB = 16
C = 8

NC, NS = 2, 16
NW = NC * NS
LANES = 128
RPC = 8
K = LANES * RPC
DP = 16 * N + 16


def _mesh():
    return plsc.VectorSubcoreMesh(
        core_axis_name="c", subcore_axis_name="s", num_cores=NC, num_subcores=NS
    )


def _make_deg_vals(EP, CPW, NE):
    def body(ver_hbm, zdeg_hbm, vals_out, vidx, dval, ones_v, deg_sh,
             lsem, dsem, hsem, wsem):
        cid = lax.axis_index("c")
        sid = lax.axis_index("s")
        wid = cid * NS + sid

        @pl.when(sid == 0)
        def _():
            pltpu.sync_copy(zdeg_hbm, deg_sh)

        def fill(l, c0):
            ones_v[pl.ds(l * 16, 16)] = jnp.full((16,), 1.0, jnp.float32)
            return c0

        lax.fori_loop(0, LANES // 16, fill, 0)
        plsc.subcore_barrier()

        NCH = NC * CPW

        def ver_hist_desc(i, b):
            r0 = sid * (NCH * RPC) + i * RPC
            return pltpu.make_async_copy(
                ver_hbm.at[pl.ds(r0, RPC)], vidx.at[b], lsem)

        def hist_descs(b):
            return [pltpu.make_async_copy(
                        ones_v, deg_sh.at[vidx.at[b, j]], hsem)
                    for j in range(RPC)]

        ver_hist_desc(0, 0).start()
        ver_hist_desc(1, 1).start()

        def hstep(i, c0):
            b3 = i % 3
            ver_hist_desc(i, b3).wait()

            @pl.when(i >= 1)
            def _():
                for d in hist_descs((i - 1) % 3):
                    d.wait()

            for d in hist_descs(b3):
                d.start(add=True)

            @pl.when(i + 2 < NCH)
            def _():
                ver_hist_desc(i + 2, (i + 2) % 3).start()

            return c0

        lax.fori_loop(0, NCH, hstep, 0)
        for d in hist_descs((NCH - 1) % 3):
            d.wait()
        plsc.subcore_barrier()

        def ver_desc(i, b):
            r0 = (wid * CPW + i) * RPC
            return pltpu.make_async_copy(
                ver_hbm.at[pl.ds(r0, RPC)], vidx.at[b], lsem)

        def dg_descs(i3, i2):
            return [pltpu.make_async_copy(
                        deg_sh.at[vidx.at[i3, j]],
                        dval.at[i2, pl.ds(j * LANES, LANES)], dsem)
                    for j in range(RPC)]

        def wdesc(i2):
            return pltpu.make_async_copy(
                dval.at[i2], vals_out.at[pl.ds(0, K)], wsem)

        ver_desc(0, 0).start()
        ver_desc(1, 1).start()
        ver_desc(0, 0).wait()
        for d in dg_descs(0, 0):
            d.start()

        def step(i, c0):
            b3 = i % 3
            b2 = i & 1
            n3 = (i + 1) % 3
            n2 = 1 - b2
            for d in dg_descs(b3, b2):
                d.wait()
            e0 = (wid * CPW + i) * K

            def rec(l, c1):
                sl = pl.ds(l * 16, 16)
                eidx = e0 + l * 16 + lax.iota(jnp.int32, 16)
                dval[b2, sl] = jnp.where(eidx < NE, 1.0 / dval[b2, sl], 0.0)
                return c1

            lax.fori_loop(0, K // 16, rec, 0)

            @pl.when(i >= 1)
            def _():
                wdesc(n2).wait()

            pltpu.async_copy(dval.at[b2], vals_out.at[pl.ds(e0, K)], wsem)

            @pl.when(i + 1 < CPW)
            def _():
                ver_desc(i + 1, n3).wait()
                for d in dg_descs(n3, n2):
                    d.start()

            @pl.when(i + 2 < CPW)
            def _():
                ver_desc(i + 2, (i + 2) % 3).start()

            return c0

        lax.fori_loop(0, CPW, step, 0)
        wdesc((CPW - 1) & 1).wait()

    return pl.kernel(
        body,
        out_type=jax.ShapeDtypeStruct((EP,), jnp.float32),
        mesh=_mesh(),
        compiler_params=pltpu.CompilerParams(use_tc_tiling_on_sc=False),
        scratch_types=[
            pltpu.VMEM((3, RPC, LANES), jnp.int32),
            pltpu.VMEM((2, K), jnp.float32),
            pltpu.VMEM((LANES,), jnp.float32),
            pltpu.VMEM_SHARED((DP,), jnp.float32),
            pltpu.SemaphoreType.DMA,
            pltpu.SemaphoreType.DMA,
            pltpu.SemaphoreType.DMA,
            pltpu.SemaphoreType.DMA,
        ],
    )


def _make_gss(EP, CPW):
    def body(table_hbm, vals_hbm, hor_hbm, sct_hbm, zh_hbm, out_hbm,
             gidx, sidx, vval, rows, h_sh, lsem, gsem, ssem):
        cid = lax.axis_index("c")
        sid = lax.axis_index("s")
        wid = cid * NS + sid

        @pl.when(sid == 0)
        def _():
            pltpu.sync_copy(zh_hbm, h_sh)

        plsc.subcore_barrier()

        def idx_descs(i, b):
            r0 = (wid * CPW + i) * RPC
            return [
                pltpu.make_async_copy(hor_hbm.at[pl.ds(r0, RPC)], gidx.at[b], lsem),
                pltpu.make_async_copy(sct_hbm.at[pl.ds(r0, RPC)], sidx.at[b], lsem),
                pltpu.make_async_copy(
                    vals_hbm.at[pl.ds(r0 * LANES, K)], vval.at[b], lsem),
            ]

        def gather_descs(i3, i2):
            return [pltpu.make_async_copy(
                        table_hbm.at[gidx.at[i3, j]],
                        rows.at[i2, pl.ds(j * LANES, LANES)], gsem)
                    for j in range(RPC)]

        def scatter_descs(i3, i2):
            return [pltpu.make_async_copy(
                        rows.at[i2, pl.ds(j * LANES, LANES)],
                        h_sh.at[sidx.at[i3, j]], ssem)
                    for j in range(RPC)]

        for d in idx_descs(0, 0):
            d.start()
        for d in idx_descs(1, 1):
            d.start()
        for d in idx_descs(0, 0):
            d.wait()
        for d in gather_descs(0, 0):
            d.start()

        def step(i, c0):
            b3 = i % 3
            b2 = i & 1
            n3 = (i + 1) % 3
            n2 = 1 - b2
            for d in gather_descs(b3, b2):
                d.wait()

            @plsc.parallel_loop(0, K // 16, unroll=4)
            def _scale(l):
                vv = vval[b3, pl.ds(l * 16, 16)]
                for t in range(16):
                    e = l * 16 + t
                    rows[b2, e, :] = rows[b2, e, :] * vv[t]

            @pl.when(i >= 1)
            def _():
                for d in scatter_descs((i - 1) % 3, n2):
                    d.wait()

            for d in scatter_descs(b3, b2):
                d.start(add=True)

            @pl.when(i + 1 < CPW)
            def _():
                for d in idx_descs(i + 1, n3):
                    d.wait()
                for d in gather_descs(n3, n2):
                    d.start()

            @pl.when(i + 2 < CPW)
            def _():
                for d in idx_descs(i + 2, (i + 2) % 3):
                    d.start()

            return c0

        lax.fori_loop(0, CPW, step, 0)
        for d in scatter_descs((CPW - 1) % 3, (CPW - 1) & 1):
            d.wait()
        plsc.subcore_barrier()

        @pl.when(sid == 0)
        def _():
            pltpu.sync_copy(h_sh, out_hbm.at[cid])

    return pl.kernel(
        body,
        out_type=jax.ShapeDtypeStruct((NC, N, EMB), jnp.float32),
        mesh=_mesh(),
        compiler_params=pltpu.CompilerParams(use_tc_tiling_on_sc=False),
        scratch_types=[
            pltpu.VMEM((3, RPC, LANES), jnp.int32),
            pltpu.VMEM((3, RPC, LANES), jnp.int32),
            pltpu.VMEM((3, K), jnp.float32),
            pltpu.VMEM((2, K, EMB), jnp.float32),
            pltpu.VMEM_SHARED((N, EMB), jnp.float32),
            pltpu.SemaphoreType.DMA,
            pltpu.SemaphoreType.DMA,
            pltpu.SemaphoreType.DMA,
        ],
    )


_BN = 2000
NP = 51200
N8 = N // 8
NP8 = NP // 8
_BB = 640


def _dense1(hAp, hBp, wp, b128, bd):
    def body(a_ref, b_ref, w_ref, b1_ref, bd_ref, out_ref):
        h = a_ref[...] + b_ref[...] + w_ref[...] + b1_ref[0]
        h = jnp.maximum(h, 0.0)
        out_ref[0] = jnp.dot(h, bd_ref[0], preferred_element_type=jnp.float32)

    return pl.pallas_call(
        body,
        grid=(NP8 // _BB, RT),
        in_specs=[
            pl.BlockSpec((_BB, 128), lambda i, r: (i, 0)),
            pl.BlockSpec((_BB, 128), lambda i, r: (i, 0)),
            pl.BlockSpec((_BB, 128), lambda i, r: (i, 0)),
            pl.BlockSpec((1, 128), lambda i, r: (0, 0)),
            pl.BlockSpec((1, 128, 128), lambda i, r: (r, 0, 0)),
        ],
        out_specs=pl.BlockSpec((1, _BB, 128), lambda i, r: (r, i, 0)),
        out_shape=jax.ShapeDtypeStruct((RT, NP8, 128), jnp.float32),
    )(hAp, hBp, wp, b128, bd)


def _final(oA, oB, selfrow, b2):
    def body(a_ref, b_ref, s_ref, b2_ref, out_ref):
        o = a_ref[...] + b_ref[...] + s_ref[...]
        out_ref[...] = o[:, :C] + b2_ref[0]

    return pl.pallas_call(
        body,
        grid=(N // _BN,),
        in_specs=[
            pl.BlockSpec((_BN, EMB), lambda i: (i, 0)),
            pl.BlockSpec((_BN, EMB), lambda i: (i, 0)),
            pl.BlockSpec((_BN, EMB), lambda i: (i, 0)),
            pl.BlockSpec((1, C), lambda i: (0, 0)),
        ],
        out_specs=pl.BlockSpec((_BN, C), lambda i: (i, 0)),
        out_shape=jax.ShapeDtypeStruct((N, C), jnp.float32),
    )(oA, oB, selfrow, b2)


def kernel(src, dst, rel, weights1, weights2, bias1, bias2):
    E = src.shape[0]
    NE = 2 * E
    EP = ((NE + NW * K - 1) // (NW * K)) * (NW * K)
    CPW = EP // (NW * K)
    EPR = EP // LANES

    src = src.astype(jnp.int32)
    dst = dst.astype(jnp.int32)
    rel = rel.astype(jnp.int32)

    ver = jnp.concatenate([rel * N + src, (rel + R) * N + dst])
    hor = jnp.concatenate([rel * N + dst, (rel + R) * N + src])
    hor2 = jnp.concatenate([rel * NP + dst, (rel + R) * NP + src])
    sct = jnp.concatenate([src, dst])
    ar = jnp.arange(EP - NE, dtype=jnp.int32)
    ver = jnp.concatenate([ver, 16 * N + (ar & 15)]).reshape(EPR, LANES)
    hor = jnp.concatenate([hor, ar & 2047]).reshape(EPR, LANES)
    hor2 = jnp.concatenate([hor2, ar & 2047]).reshape(EPR, LANES)
    sct = jnp.concatenate([sct, ar & 7]).reshape(EPR, LANES)

    zdeg = jnp.zeros((DP,), jnp.float32)
    zh = jnp.zeros((N, EMB), jnp.float32)

    vals = _make_deg_vals(EP, CPW, NE)(ver, zdeg)

    w1f = weights1.reshape(RT * N, EMB)
    hpart = _make_gss(EP, CPW)(w1f, vals, hor, sct, zh)

    hAp = jnp.pad(hpart[0].reshape(N8, 128), ((0, NP8 - N8), (0, 0)))
    hBp = jnp.pad(hpart[1].reshape(N8, 128), ((0, NP8 - N8), (0, 0)))
    wp = jnp.pad(weights1[2 * R].reshape(N8, 128), ((0, NP8 - N8), (0, 0)))
    b128 = jnp.tile(bias1, 8).reshape(1, 128)
    w2p = jnp.pad(weights2, ((0, 0), (0, 0), (0, EMB - C)))
    bd = jnp.einsum('kl,rec->rkelc', jnp.eye(8, dtype=jnp.float32),
                    w2p).reshape(RT, 128, 128)

    hw2L = _dense1(hAp, hBp, wp, b128, bd)

    opart = _make_gss(EP, CPW)(hw2L.reshape(RT * NP, EMB), vals, hor2, sct, zh)

    selfrow = hw2L[2 * R, :N8].reshape(N, EMB)
    return _final(opart[0], opart[1], selfrow, bias2.reshape(1, C))


# speedup vs baseline: 1.0775x; 1.0775x over previous
import jax
import jax.numpy as jnp
from jax import lax
from jax.experimental import pallas as pl
from jax.experimental.pallas import tpu as pltpu
from jax.experimental.pallas import tpu_sc as plsc

N = 50000
R = 8
RT = 2 * R + 1
EMB = 16
C = 8

NC, NS = 2, 16
NW = NC * NS
LANES = 128
RPC = 8
K = LANES * RPC
DP = 16 * N + 16


def _mesh():
    return plsc.VectorSubcoreMesh(
        core_axis_name="c", subcore_axis_name="s", num_cores=NC, num_subcores=NS
    )


def _make_deg_vals(EP, CPW, NE):
    def body(ver_hbm, zdeg_hbm, vals_out, vidx, dval, ones_v, deg_sh,
             lsem, dsem, hsem, wsem):
        cid = lax.axis_index("c")
        sid = lax.axis_index("s")
        wid = cid * NS + sid

        @pl.when(sid == 0)
        def _():
            pltpu.sync_copy(zdeg_hbm, deg_sh)

        def fill(l, c0):
            ones_v[pl.ds(l * 16, 16)] = jnp.full((16,), 1.0, jnp.float32)
            return c0

        lax.fori_loop(0, LANES // 16, fill, 0)
        plsc.subcore_barrier()

        NCH = NC * CPW

        def ver_hist_desc(i, b):
            r0 = sid * (NCH * RPC) + i * RPC
            return pltpu.make_async_copy(
                ver_hbm.at[pl.ds(r0, RPC)], vidx.at[b], lsem)

        def hist_descs(b):
            return [pltpu.make_async_copy(
                        ones_v, deg_sh.at[vidx.at[b, j]], hsem)
                    for j in range(RPC)]

        ver_hist_desc(0, 0).start()
        ver_hist_desc(1, 1).start()

        def hstep(i, c0):
            b3 = i % 3
            ver_hist_desc(i, b3).wait()

            @pl.when(i >= 1)
            def _():
                for d in hist_descs((i - 1) % 3):
                    d.wait()

            for d in hist_descs(b3):
                d.start(add=True)

            @pl.when(i + 2 < NCH)
            def _():
                ver_hist_desc(i + 2, (i + 2) % 3).start()

            return c0

        lax.fori_loop(0, NCH, hstep, 0)
        for d in hist_descs((NCH - 1) % 3):
            d.wait()
        plsc.subcore_barrier()

        def ver_desc(i, b):
            r0 = (wid * CPW + i) * RPC
            return pltpu.make_async_copy(
                ver_hbm.at[pl.ds(r0, RPC)], vidx.at[b], lsem)

        def dg_descs(i3, i2):
            return [pltpu.make_async_copy(
                        deg_sh.at[vidx.at[i3, j]],
                        dval.at[i2, pl.ds(j * LANES, LANES)], dsem)
                    for j in range(RPC)]

        def wdesc(i2):
            return pltpu.make_async_copy(
                dval.at[i2], vals_out.at[pl.ds(0, K)], wsem)

        ver_desc(0, 0).start()
        ver_desc(1, 1).start()
        ver_desc(0, 0).wait()
        for d in dg_descs(0, 0):
            d.start()

        def step(i, c0):
            b3 = i % 3
            b2 = i & 1
            n3 = (i + 1) % 3
            n2 = 1 - b2
            for d in dg_descs(b3, b2):
                d.wait()
            e0 = (wid * CPW + i) * K

            def rec(l, c1):
                sl = pl.ds(l * 16, 16)
                eidx = e0 + l * 16 + lax.iota(jnp.int32, 16)
                dval[b2, sl] = jnp.where(eidx < NE, 1.0 / dval[b2, sl], 0.0)
                return c1

            lax.fori_loop(0, K // 16, rec, 0)

            @pl.when(i >= 1)
            def _():
                wdesc(n2).wait()

            pltpu.async_copy(dval.at[b2], vals_out.at[pl.ds(e0, K)], wsem)

            @pl.when(i + 1 < CPW)
            def _():
                ver_desc(i + 1, n3).wait()
                for d in dg_descs(n3, n2):
                    d.start()

            @pl.when(i + 2 < CPW)
            def _():
                ver_desc(i + 2, (i + 2) % 3).start()

            return c0

        lax.fori_loop(0, CPW, step, 0)
        wdesc((CPW - 1) & 1).wait()

    return pl.kernel(
        body,
        out_type=jax.ShapeDtypeStruct((EP,), jnp.float32),
        mesh=_mesh(),
        compiler_params=pltpu.CompilerParams(use_tc_tiling_on_sc=False),
        scratch_types=[
            pltpu.VMEM((3, RPC, LANES), jnp.int32),
            pltpu.VMEM((2, K), jnp.float32),
            pltpu.VMEM((LANES,), jnp.float32),
            pltpu.VMEM_SHARED((DP,), jnp.float32),
            pltpu.SemaphoreType.DMA,
            pltpu.SemaphoreType.DMA,
            pltpu.SemaphoreType.DMA,
            pltpu.SemaphoreType.DMA,
        ],
    )


def _make_gss(EP, CPW):
    def body(table_hbm, vals_hbm, hor_hbm, sct_hbm, zh_hbm, out_hbm,
             gidx, sidx, vval, rows, h_sh, lsem, gsem, ssem):
        cid = lax.axis_index("c")
        sid = lax.axis_index("s")
        wid = cid * NS + sid

        @pl.when(sid == 0)
        def _():
            pltpu.sync_copy(zh_hbm, h_sh)

        plsc.subcore_barrier()

        def idx_descs(i, b):
            r0 = (wid * CPW + i) * RPC
            return [
                pltpu.make_async_copy(hor_hbm.at[pl.ds(r0, RPC)], gidx.at[b], lsem),
                pltpu.make_async_copy(sct_hbm.at[pl.ds(r0, RPC)], sidx.at[b], lsem),
                pltpu.make_async_copy(
                    vals_hbm.at[pl.ds(r0 * LANES, K)], vval.at[b], lsem),
            ]

        def gather_descs(i3, i2):
            return [pltpu.make_async_copy(
                        table_hbm.at[gidx.at[i3, j]],
                        rows.at[i2, pl.ds(j * LANES, LANES)], gsem)
                    for j in range(RPC)]

        def scatter_descs(i3, i2):
            return [pltpu.make_async_copy(
                        rows.at[i2, pl.ds(j * LANES, LANES)],
                        h_sh.at[sidx.at[i3, j]], ssem)
                    for j in range(RPC)]

        for d in idx_descs(0, 0):
            d.start()
        for d in idx_descs(1, 1):
            d.start()
        for d in idx_descs(0, 0):
            d.wait()
        for d in gather_descs(0, 0):
            d.start()

        def step(i, c0):
            b3 = i % 3
            b2 = i & 1
            n3 = (i + 1) % 3
            n2 = 1 - b2
            for d in gather_descs(b3, b2):
                d.wait()

            def scale(l, c1):
                vv = vval[b3, pl.ds(l * 16, 16)]
                for t in range(16):
                    e = l * 16 + t
                    rows[b2, e, :] = rows[b2, e, :] * vv[t]
                return c1

            lax.fori_loop(0, K // 16, scale, 0)

            @pl.when(i >= 1)
            def _():
                for d in scatter_descs((i - 1) % 3, n2):
                    d.wait()

            for d in scatter_descs(b3, b2):
                d.start(add=True)

            @pl.when(i + 1 < CPW)
            def _():
                for d in idx_descs(i + 1, n3):
                    d.wait()
                for d in gather_descs(n3, n2):
                    d.start()

            @pl.when(i + 2 < CPW)
            def _():
                for d in idx_descs(i + 2, (i + 2) % 3):
                    d.start()

            return c0

        lax.fori_loop(0, CPW, step, 0)
        for d in scatter_descs((CPW - 1) % 3, (CPW - 1) & 1):
            d.wait()
        plsc.subcore_barrier()

        @pl.when(sid == 0)
        def _():
            pltpu.sync_copy(h_sh, out_hbm.at[cid])

    return pl.kernel(
        body,
        out_type=jax.ShapeDtypeStruct((NC, N, EMB), jnp.float32),
        mesh=_mesh(),
        compiler_params=pltpu.CompilerParams(use_tc_tiling_on_sc=False),
        scratch_types=[
            pltpu.VMEM((3, RPC, LANES), jnp.int32),
            pltpu.VMEM((3, RPC, LANES), jnp.int32),
            pltpu.VMEM((3, K), jnp.float32),
            pltpu.VMEM((2, K, EMB), jnp.float32),
            pltpu.VMEM_SHARED((N, EMB), jnp.float32),
            pltpu.SemaphoreType.DMA,
            pltpu.SemaphoreType.DMA,
            pltpu.SemaphoreType.DMA,
        ],
    )


_BN = 2000
NP = 51200
N8 = N // 8
NP8 = NP // 8
_BB = 640


def _dense1(hAp, hBp, wp, b128, bd):
    def body(a_ref, b_ref, w_ref, b1_ref, bd_ref, out_ref):
        h = a_ref[...] + b_ref[...] + w_ref[...] + b1_ref[0]
        h = jnp.maximum(h, 0.0)
        out_ref[0] = jnp.dot(h, bd_ref[0], preferred_element_type=jnp.float32)

    return pl.pallas_call(
        body,
        grid=(NP8 // _BB, RT),
        in_specs=[
            pl.BlockSpec((_BB, 128), lambda i, r: (i, 0)),
            pl.BlockSpec((_BB, 128), lambda i, r: (i, 0)),
            pl.BlockSpec((_BB, 128), lambda i, r: (i, 0)),
            pl.BlockSpec((1, 128), lambda i, r: (0, 0)),
            pl.BlockSpec((1, 128, 128), lambda i, r: (r, 0, 0)),
        ],
        out_specs=pl.BlockSpec((1, _BB, 128), lambda i, r: (r, i, 0)),
        out_shape=jax.ShapeDtypeStruct((RT, NP8, 128), jnp.float32),
    )(hAp, hBp, wp, b128, bd)


def _final(oA, oB, selfrow, b2):
    def body(a_ref, b_ref, s_ref, b2_ref, out_ref):
        o = a_ref[...] + b_ref[...] + s_ref[...]
        out_ref[...] = o[:, :C] + b2_ref[0]

    return pl.pallas_call(
        body,
        grid=(N // _BN,),
        in_specs=[
            pl.BlockSpec((_BN, EMB), lambda i: (i, 0)),
            pl.BlockSpec((_BN, EMB), lambda i: (i, 0)),
            pl.BlockSpec((_BN, EMB), lambda i: (i, 0)),
            pl.BlockSpec((1, C), lambda i: (0, 0)),
        ],
        out_specs=pl.BlockSpec((_BN, C), lambda i: (i, 0)),
        out_shape=jax.ShapeDtypeStruct((N, C), jnp.float32),
    )(oA, oB, selfrow, b2)


def kernel(src, dst, rel, weights1, weights2, bias1, bias2):
    E = src.shape[0]
    NE = 2 * E
    EP = ((NE + NW * K - 1) // (NW * K)) * (NW * K)
    CPW = EP // (NW * K)
    EPR = EP // LANES

    src = src.astype(jnp.int32)
    dst = dst.astype(jnp.int32)
    rel = rel.astype(jnp.int32)

    ver = jnp.concatenate([rel * N + src, (rel + R) * N + dst])
    hor = jnp.concatenate([rel * N + dst, (rel + R) * N + src])
    hor2 = jnp.concatenate([rel * NP + dst, (rel + R) * NP + src])
    sct = jnp.concatenate([src, dst])
    ar = jnp.arange(EP - NE, dtype=jnp.int32)
    ver = jnp.concatenate([ver, 16 * N + (ar & 15)]).reshape(EPR, LANES)
    hor = jnp.concatenate([hor, ar & 2047]).reshape(EPR, LANES)
    hor2 = jnp.concatenate([hor2, ar & 2047]).reshape(EPR, LANES)
    sct = jnp.concatenate([sct, ar & 7]).reshape(EPR, LANES)

    zdeg = jnp.zeros((DP,), jnp.float32)
    zh = jnp.zeros((N, EMB), jnp.float32)

    vals = _make_deg_vals(EP, CPW, NE)(ver, zdeg)

    w1f = weights1.reshape(RT * N, EMB)
    hpart = _make_gss(EP, CPW)(w1f, vals, hor, sct, zh)

    hAp = jnp.pad(hpart[0].reshape(N8, 128), ((0, NP8 - N8), (0, 0)))
    hBp = jnp.pad(hpart[1].reshape(N8, 128), ((0, NP8 - N8), (0, 0)))
    wp = jnp.pad(weights1[2 * R].reshape(N8, 128), ((0, NP8 - N8), (0, 0)))
    b128 = jnp.tile(bias1, 8).reshape(1, 128)
    w2p = jnp.pad(weights2, ((0, 0), (0, 0), (0, EMB - C)))
    bd = jnp.einsum('kl,rec->rkelc', jnp.eye(8, dtype=jnp.float32),
                    w2p).reshape(RT, 128, 128)

    hw2L = _dense1(hAp, hBp, wp, b128, bd)

    opart = _make_gss(EP, CPW)(hw2L.reshape(RT * NP, EMB), vals, hor2, sct, zh)

    selfrow = hw2L[2 * R, :N8].reshape(N, EMB)
    return _final(opart[0], opart[1], selfrow, bias2.reshape(1, C))
